# baseline (speedup 1.0000x reference)
import functools

import jax
import jax.numpy as jnp
from jax.experimental import pallas as pl
from jax.experimental.pallas import tpu as pltpu

K = 5
EPS = 1e-8
N = 4096
D = 128
BCOL = 2048
RCHUNK = 256
NCHAIN = 4
NEG = -3.0e38


def _insert(tops, v):
    new = []
    for i, t in enumerate(tops):
        hi = jnp.maximum(t, v)
        if i + 1 < len(tops):
            v = jnp.minimum(t, v)
        new.append(hi)
    return new


def _merge(a, b):
    k = len(a)
    out = []
    for i in range(k):
        terms = [a[i], b[i]]
        for j in range(i):
            terms.append(jnp.minimum(a[j], b[i - 1 - j]))
        m = terms[0]
        for t in terms[1:]:
            m = jnp.maximum(m, t)
        out.append(m)
    return out


def _knn_loss_kernel(s_ref, t_ref, out_ref, acc_ref, tbf_ref):
    i = pl.program_id(0)
    nsteps = pl.num_programs(0)

    @pl.when(i == 0)
    def _cast_teacher():
        tbf_ref[...] = t_ref[...].astype(jnp.bfloat16)

    s = s_ref[...].astype(jnp.bfloat16)
    chains = [[jnp.full((8, BCOL), jnp.bfloat16(NEG))] * K
              for _ in range(NCHAIN)]

    for c in range(N // RCHUNK):
        t = tbf_ref[pl.ds(c * RCHUNK, RCHUNK), :]
        x = jax.lax.dot_general(
            t, s, (((1,), (1,)), ((), ())),
            preferred_element_type=jnp.float32,
        ).astype(jnp.bfloat16)
        for j in range(RCHUNK // 8):
            v = x[j * 8:(j + 1) * 8, :]
            k = j % NCHAIN
            chains[k] = _insert(chains[k], v)

    while len(chains) > 1:
        chains = [_merge(chains[i2], chains[i2 + 1])
                  for i2 in range(0, len(chains), 2)]
    tops = chains[0]

    for half in (4, 2, 1):
        a = [t[:half, :] for t in tops]
        b = [t[half:, :] for t in tops]
        tops = _merge(a, b)

    total = tops[0].astype(jnp.float32)
    for t in tops[1:]:
        total = total + t.astype(jnp.float32)
    mean5 = total * jnp.float32(1.0 / K)
    partial = jnp.sum(jnp.log(mean5 + jnp.float32(EPS)))

    @pl.when(i == 0)
    def _init():
        acc_ref[0] = jnp.float32(0.0)

    acc_ref[0] = acc_ref[0] + partial

    @pl.when(i == nsteps - 1)
    def _fin():
        out_ref[0] = -acc_ref[0] * jnp.float32(1.0 / N)


@functools.partial(jax.jit, static_argnames=("interpret",))
def kernel(student_output, teacher_output, interpret=False):
    nsteps = N // BCOL
    out = pl.pallas_call(
        _knn_loss_kernel,
        grid=(nsteps,),
        in_specs=[
            pl.BlockSpec((BCOL, D), lambda i: (i, 0)),
            pl.BlockSpec((N, D), lambda i: (0, 0)),
        ],
        out_specs=pl.BlockSpec(memory_space=pltpu.SMEM),
        out_shape=jax.ShapeDtypeStruct((1,), jnp.float32),
        scratch_shapes=[
            pltpu.SMEM((1,), jnp.float32),
            pltpu.VMEM((N, D), jnp.bfloat16),
        ],
        interpret=interpret,
    )(student_output, teacher_output)
    return jnp.reshape(out, ())


# speedup vs baseline: 1.0552x; 1.0552x over previous
import functools

import jax
import jax.numpy as jnp
from jax.experimental import pallas as pl
from jax.experimental.pallas import tpu as pltpu

K = 5
EPS = 1e-8
N = 4096
D = 128
BCOL = 2048
RCHUNK = 1024
NCHAIN = 4
NEG = -3.0e38


def _insert(tops, v):
    new = []
    for i, t in enumerate(tops):
        hi = jnp.maximum(t, v)
        if i + 1 < len(tops):
            v = jnp.minimum(t, v)
        new.append(hi)
    return new


def _merge(a, b):
    k = len(a)
    out = []
    for i in range(k):
        terms = [a[i], b[i]]
        for j in range(i):
            terms.append(jnp.minimum(a[j], b[i - 1 - j]))
        m = terms[0]
        for t in terms[1:]:
            m = jnp.maximum(m, t)
        out.append(m)
    return out


def _knn_loss_kernel(s_ref, t_ref, out_ref, acc_ref, tbf_ref):
    i = pl.program_id(0)
    nsteps = pl.num_programs(0)

    @pl.when(i == 0)
    def _cast_teacher():
        tbf_ref[...] = t_ref[...].astype(jnp.bfloat16)

    s = s_ref[...].astype(jnp.bfloat16)
    chains = [[jnp.full((8, BCOL), jnp.bfloat16(NEG))] * K
              for _ in range(NCHAIN)]

    for c in range(N // RCHUNK):
        t = tbf_ref[pl.ds(c * RCHUNK, RCHUNK), :]
        x = jax.lax.dot_general(
            t, s, (((1,), (1,)), ((), ())),
            preferred_element_type=jnp.float32,
        ).astype(jnp.bfloat16)
        for j in range(RCHUNK // 8):
            v = x[j * 8:(j + 1) * 8, :]
            k = j % NCHAIN
            chains[k] = _insert(chains[k], v)

    while len(chains) > 1:
        chains = [_merge(chains[i2], chains[i2 + 1])
                  for i2 in range(0, len(chains), 2)]
    tops = chains[0]

    for half in (4, 2, 1):
        a = [t[:half, :] for t in tops]
        b = [t[half:, :] for t in tops]
        tops = _merge(a, b)

    total = tops[0].astype(jnp.float32)
    for t in tops[1:]:
        total = total + t.astype(jnp.float32)
    mean5 = total * jnp.float32(1.0 / K)
    partial = jnp.sum(jnp.log(mean5 + jnp.float32(EPS)))

    @pl.when(i == 0)
    def _init():
        acc_ref[0] = jnp.float32(0.0)

    acc_ref[0] = acc_ref[0] + partial

    @pl.when(i == nsteps - 1)
    def _fin():
        out_ref[0] = -acc_ref[0] * jnp.float32(1.0 / N)


@functools.partial(jax.jit, static_argnames=("interpret",))
def kernel(student_output, teacher_output, interpret=False):
    nsteps = N // BCOL
    out = pl.pallas_call(
        _knn_loss_kernel,
        grid=(nsteps,),
        in_specs=[
            pl.BlockSpec((BCOL, D), lambda i: (i, 0)),
            pl.BlockSpec((N, D), lambda i: (0, 0)),
        ],
        out_specs=pl.BlockSpec(memory_space=pltpu.SMEM),
        out_shape=jax.ShapeDtypeStruct((1,), jnp.float32),
        scratch_shapes=[
            pltpu.SMEM((1,), jnp.float32),
            pltpu.VMEM((N, D), jnp.bfloat16),
        ],
        interpret=interpret,
    )(student_output, teacher_output)
    return jnp.reshape(out, ())


# speedup vs baseline: 1.0586x; 1.0032x over previous
import functools

import jax
import jax.numpy as jnp
from jax.experimental import pallas as pl
from jax.experimental.pallas import tpu as pltpu

K = 5
EPS = 1e-8
N = 4096
D = 128
BCOL = 2048
RCHUNK = 2048
NCHAIN = 4
NEG = -3.0e38


def _insert(tops, v):
    new = []
    for i, t in enumerate(tops):
        hi = jnp.maximum(t, v)
        if i + 1 < len(tops):
            v = jnp.minimum(t, v)
        new.append(hi)
    return new


def _merge(a, b):
    k = len(a)
    out = []
    for i in range(k):
        terms = [a[i], b[i]]
        for j in range(i):
            terms.append(jnp.minimum(a[j], b[i - 1 - j]))
        m = terms[0]
        for t in terms[1:]:
            m = jnp.maximum(m, t)
        out.append(m)
    return out


def _knn_loss_kernel(s_ref, t_ref, out_ref, acc_ref, tbf_ref):
    i = pl.program_id(0)
    nsteps = pl.num_programs(0)

    @pl.when(i == 0)
    def _cast_teacher():
        tbf_ref[...] = t_ref[...].astype(jnp.bfloat16)

    s = s_ref[...].astype(jnp.bfloat16)
    chains = [[jnp.full((8, BCOL), jnp.bfloat16(NEG))] * K
              for _ in range(NCHAIN)]

    for c in range(N // RCHUNK):
        t = tbf_ref[pl.ds(c * RCHUNK, RCHUNK), :]
        x = jax.lax.dot_general(
            t, s, (((1,), (1,)), ((), ())),
            preferred_element_type=jnp.float32,
        ).astype(jnp.bfloat16)
        for j in range(RCHUNK // 8):
            v = x[j * 8:(j + 1) * 8, :]
            k = j % NCHAIN
            chains[k] = _insert(chains[k], v)

    while len(chains) > 1:
        chains = [_merge(chains[i2], chains[i2 + 1])
                  for i2 in range(0, len(chains), 2)]
    tops = chains[0]

    for half in (4, 2, 1):
        a = [t[:half, :] for t in tops]
        b = [t[half:, :] for t in tops]
        tops = _merge(a, b)

    total = tops[0].astype(jnp.float32)
    for t in tops[1:]:
        total = total + t.astype(jnp.float32)
    mean5 = total * jnp.float32(1.0 / K)
    partial = jnp.sum(jnp.log(mean5 + jnp.float32(EPS)))

    @pl.when(i == 0)
    def _init():
        acc_ref[0] = jnp.float32(0.0)

    acc_ref[0] = acc_ref[0] + partial

    @pl.when(i == nsteps - 1)
    def _fin():
        out_ref[0] = -acc_ref[0] * jnp.float32(1.0 / N)


@functools.partial(jax.jit, static_argnames=("interpret",))
def kernel(student_output, teacher_output, interpret=False):
    nsteps = N // BCOL
    out = pl.pallas_call(
        _knn_loss_kernel,
        grid=(nsteps,),
        in_specs=[
            pl.BlockSpec((BCOL, D), lambda i: (i, 0)),
            pl.BlockSpec((N, D), lambda i: (0, 0)),
        ],
        out_specs=pl.BlockSpec(memory_space=pltpu.SMEM),
        out_shape=jax.ShapeDtypeStruct((1,), jnp.float32),
        scratch_shapes=[
            pltpu.SMEM((1,), jnp.float32),
            pltpu.VMEM((N, D), jnp.bfloat16),
        ],
        interpret=interpret,
    )(student_output, teacher_output)
    return jnp.reshape(out, ())


# speedup vs baseline: 1.0602x; 1.0015x over previous
import functools

import jax
import jax.numpy as jnp
from jax.experimental import pallas as pl
from jax.experimental.pallas import tpu as pltpu

K = 5
EPS = 1e-8
N = 4096
D = 128
BCOL = 2048
RCHUNK = 4096
NCHAIN = 4
NEG = -3.0e38


def _insert(tops, v):
    new = []
    for i, t in enumerate(tops):
        hi = jnp.maximum(t, v)
        if i + 1 < len(tops):
            v = jnp.minimum(t, v)
        new.append(hi)
    return new


def _merge(a, b):
    k = len(a)
    out = []
    for i in range(k):
        terms = [a[i], b[i]]
        for j in range(i):
            terms.append(jnp.minimum(a[j], b[i - 1 - j]))
        m = terms[0]
        for t in terms[1:]:
            m = jnp.maximum(m, t)
        out.append(m)
    return out


def _knn_loss_kernel(s_ref, t_ref, out_ref, acc_ref, tbf_ref):
    i = pl.program_id(0)
    nsteps = pl.num_programs(0)

    @pl.when(i == 0)
    def _cast_teacher():
        tbf_ref[...] = t_ref[...].astype(jnp.bfloat16)

    s = s_ref[...].astype(jnp.bfloat16)
    chains = [[jnp.full((8, BCOL), jnp.bfloat16(NEG))] * K
              for _ in range(NCHAIN)]

    for c in range(N // RCHUNK):
        t = tbf_ref[pl.ds(c * RCHUNK, RCHUNK), :]
        x = jax.lax.dot_general(
            t, s, (((1,), (1,)), ((), ())),
            preferred_element_type=jnp.float32,
        ).astype(jnp.bfloat16)
        for j in range(RCHUNK // 8):
            v = x[j * 8:(j + 1) * 8, :]
            k = j % NCHAIN
            chains[k] = _insert(chains[k], v)

    while len(chains) > 1:
        chains = [_merge(chains[i2], chains[i2 + 1])
                  for i2 in range(0, len(chains), 2)]
    tops = chains[0]

    for half in (4, 2, 1):
        a = [t[:half, :] for t in tops]
        b = [t[half:, :] for t in tops]
        tops = _merge(a, b)

    total = tops[0].astype(jnp.float32)
    for t in tops[1:]:
        total = total + t.astype(jnp.float32)
    mean5 = total * jnp.float32(1.0 / K)
    partial = jnp.sum(jnp.log(mean5 + jnp.float32(EPS)))

    @pl.when(i == 0)
    def _init():
        acc_ref[0] = jnp.float32(0.0)

    acc_ref[0] = acc_ref[0] + partial

    @pl.when(i == nsteps - 1)
    def _fin():
        out_ref[0] = -acc_ref[0] * jnp.float32(1.0 / N)


@functools.partial(jax.jit, static_argnames=("interpret",))
def kernel(student_output, teacher_output, interpret=False):
    nsteps = N // BCOL
    out = pl.pallas_call(
        _knn_loss_kernel,
        grid=(nsteps,),
        in_specs=[
            pl.BlockSpec((BCOL, D), lambda i: (i, 0)),
            pl.BlockSpec((N, D), lambda i: (0, 0)),
        ],
        out_specs=pl.BlockSpec(memory_space=pltpu.SMEM),
        out_shape=jax.ShapeDtypeStruct((1,), jnp.float32),
        scratch_shapes=[
            pltpu.SMEM((1,), jnp.float32),
            pltpu.VMEM((N, D), jnp.bfloat16),
        ],
        interpret=interpret,
    )(student_output, teacher_output)
    return jnp.reshape(out, ())
